# baseline (speedup 1.0000x reference)
import functools

import jax
import jax.numpy as jnp
from jax import lax
from jax.experimental import pallas as pl
from jax.experimental.pallas import tpu as pltpu
from jax.experimental.pallas import tpu_sc as plsc

N = 10000
M = 128
E = 320000
NLAB = 2048
NC, NS = 2, 16
NW = NC * NS
C = 512
NBUF = 4
K = NBUF * (-(-E // (NW * C * NBUF)))
EPAD = NW * K * C
NPAD = 10112
DUMMY = 10000
RPT = NPAD // NS
W = 16
KL = NLAB // NW

_mesh = plsc.VectorSubcoreMesh(
    core_axis_name="c", subcore_axis_name="s", num_cores=NC, num_subcores=NS)


@functools.partial(
    pl.kernel,
    out_type=jax.ShapeDtypeStruct((NC, NPAD, W), jnp.float32),
    mesh=_mesh,
    scratch_types=[
        pltpu.VMEM((K, C), jnp.int32),
        pltpu.VMEM((K, C), jnp.int32),
        pltpu.VMEM((NBUF, C, W), jnp.float32),
        pltpu.VMEM_SHARED((NPAD, W), jnp.float32),
        [pltpu.SemaphoreType.DMA] * NBUF,
    ],
    compiler_params=pltpu.CompilerParams(use_tc_tiling_on_sc=False),
)
def _segsum_sc(p_hbm, srcs_hbm, dsts_hbm, zeros_hbm, out_hbm,
               src_v, dst_v, rows_v, agg_sh, sems):
    cid = lax.axis_index("c")
    sid = lax.axis_index("s")
    wid = sid * NC + cid
    pltpu.sync_copy(srcs_hbm.at[wid], src_v)
    pltpu.sync_copy(dsts_hbm.at[wid], dst_v)
    for b in range(NBUF):
        pltpu.async_copy(p_hbm.at[src_v.at[b]], rows_v.at[b], sems[b])
    pltpu.sync_copy(zeros_hbm.at[pl.ds(sid * RPT, RPT)],
                    agg_sh.at[pl.ds(sid * RPT, RPT)])
    plsc.subcore_barrier()

    def body(g, carry):
        for b in range(NBUF):
            k = g * NBUF + b
            pltpu.make_async_copy(
                p_hbm.at[src_v.at[k]], rows_v.at[b], sems[b]).wait()
            pltpu.sync_copy(rows_v.at[b], agg_sh.at[dst_v.at[k]], add=True)

            @pl.when(k + NBUF < K)
            def _():
                pltpu.async_copy(
                    p_hbm.at[src_v.at[k + NBUF]], rows_v.at[b], sems[b])
        return carry

    lax.fori_loop(0, K // NBUF, body, 0)
    plsc.subcore_barrier()
    pltpu.sync_copy(agg_sh.at[pl.ds(sid * RPT, RPT)],
                    out_hbm.at[cid, pl.ds(sid * RPT, RPT)])


@functools.partial(
    pl.kernel,
    out_type=jax.ShapeDtypeStruct((NLAB, W), jnp.float32),
    mesh=_mesh,
    scratch_types=[
        pltpu.VMEM((1, KL), jnp.int32),
        pltpu.VMEM((KL, W), jnp.float32),
        pltpu.SemaphoreType.DMA,
    ],
    compiler_params=pltpu.CompilerParams(use_tc_tiling_on_sc=False),
)
def _label_gather_sc(h2_hbm, lbl_hbm, out_hbm, idx_v, rows_v, sem):
    cid = lax.axis_index("c")
    sid = lax.axis_index("s")
    wid = sid * NC + cid
    pltpu.sync_copy(lbl_hbm.at[wid], idx_v)
    pltpu.async_copy(h2_hbm.at[idx_v.at[0]], rows_v, sem).wait()
    pltpu.sync_copy(rows_v, out_hbm.at[pl.ds(wid * KL, KL)])


def _tc0_body(score_ref, wa_ref, wb_ref, sa_ref, p_ref):
    s = score_ref[:]
    sa_ref[:] = jnp.dot(s, wa_ref[:], preferred_element_type=jnp.float32)
    p_ref[:] = jnp.dot(s, wb_ref[:], preferred_element_type=jnp.float32)


def _tc_mid_body(sa_ref, agg_ref, b1_ref, w2_ref, b2_ref, w3_ref, b3_ref,
                 wa_ref, wb_ref, sa_o, p_o):
    agg = agg_ref[0, :N, :] + agg_ref[1, :N, :]
    h = jnp.maximum(sa_ref[:] + agg + b1_ref[:], 0.0)
    h = jnp.maximum(
        jnp.dot(h, w2_ref[:], preferred_element_type=jnp.float32) + b2_ref[:],
        0.0)
    s = jnp.dot(h, w3_ref[:], preferred_element_type=jnp.float32) + b3_ref[:]
    sa_o[:] = jnp.dot(s, wa_ref[:], preferred_element_type=jnp.float32)
    p_o[:] = jnp.dot(s, wb_ref[:], preferred_element_type=jnp.float32)


def _tc_last_body(sa_ref, agg_ref, b1_ref, w2_ref, b2_ref, h2_o):
    agg = agg_ref[0, :N, :] + agg_ref[1, :N, :]
    h = jnp.maximum(sa_ref[:] + agg + b1_ref[:], 0.0)
    h2_o[:] = jnp.maximum(
        jnp.dot(h, w2_ref[:], preferred_element_type=jnp.float32) + b2_ref[:],
        0.0)


def _tc_fin_body(hl_ref, w3_ref, b3_ref, g1_ref, g1b_ref, g2_ref, g2b_ref,
                 g3_ref, g3b_ref, out_ref):
    s = jnp.dot(hl_ref[:], w3_ref[:], preferred_element_type=jnp.float32)
    s = s + b3_ref[:]
    h = jnp.maximum(
        jnp.dot(s, g1_ref[:], preferred_element_type=jnp.float32) + g1b_ref[:],
        0.0)
    h = jnp.maximum(
        jnp.dot(h, g2_ref[:], preferred_element_type=jnp.float32) + g2b_ref[:],
        0.0)
    lg = jnp.dot(h, g3_ref[:], preferred_element_type=jnp.float32) + g3b_ref[:]
    out_ref[:] = 1.0 / (1.0 + jnp.exp(-lg))


def _f32(shape):
    return jax.ShapeDtypeStruct(shape, jnp.float32)


_tc0 = pl.pallas_call(_tc0_body, out_shape=(_f32((N, W)), _f32((N, W))))
_tc_mid = pl.pallas_call(_tc_mid_body, out_shape=(_f32((N, W)), _f32((N, W))))
_tc_last = pl.pallas_call(_tc_last_body, out_shape=_f32((N, W)))
_tc_fin = pl.pallas_call(_tc_fin_body, out_shape=_f32((NLAB, 1)))


def kernel(score, edges, label_idx, W1, b1, W2, b2, W3, b3,
           G1, g1, G2, g2, G3, g3):
    src = edges[0].astype(jnp.int32)
    dst = edges[1].astype(jnp.int32)
    srcs = jnp.concatenate(
        [src, jnp.zeros((EPAD - E,), jnp.int32)]).reshape(NW, K, C)
    dsts = jnp.concatenate(
        [dst, jnp.full((EPAD - E,), DUMMY, jnp.int32)]).reshape(NW, K, C)
    lbl = label_idx.astype(jnp.int32).reshape(NW, 1, KL)
    zblk = jnp.zeros((NPAD, W), jnp.float32)

    w1a = jnp.pad(W1[:M], ((0, 0), (0, W - 7)))
    w1b = jnp.pad(W1[M:], ((0, 0), (0, W - 7)))
    b1p = jnp.pad(b1, (0, W - 7)).reshape(1, W)
    w2p = jnp.pad(W2, ((0, W - 7), (0, W - 7)))
    b2p = jnp.pad(b2, (0, W - 7)).reshape(1, W)
    w3p = jnp.pad(W3, ((0, W - 7), (0, 0)))
    b3p = b3.reshape(1, M)
    g1p = jnp.pad(G1, ((0, 0), (0, W - 9)))
    g1bp = jnp.pad(g1, (0, W - 9)).reshape(1, W)
    g2p = jnp.pad(G2, ((0, W - 9), (0, W - 9)))
    g2bp = jnp.pad(g2, (0, W - 9)).reshape(1, W)
    g3p = jnp.pad(G3, ((0, W - 9), (0, 0)))
    g3bp = g3.reshape(1, 1)

    sa, p = _tc0(score, w1a, w1b)
    h2 = None
    for t in range(3):
        agg = _segsum_sc(p, srcs, dsts, zblk)
        if t < 2:
            sa, p = _tc_mid(sa, agg, b1p, w2p, b2p, w3p, b3p, w1a, w1b)
        else:
            h2 = _tc_last(sa, agg, b1p, w2p, b2p)

    hl = _label_gather_sc(h2, lbl)
    return _tc_fin(hl, w3p, b3p, g1p, g1bp, g2p, g2bp, g3p, g3bp)


# speedup vs baseline: 1.4773x; 1.4773x over previous
import functools

import jax
import jax.numpy as jnp
from jax import lax
from jax.experimental import pallas as pl
from jax.experimental.pallas import tpu as pltpu
from jax.experimental.pallas import tpu_sc as plsc

N = 10000
M = 128
E = 320000
NLAB = 2048
NC, NS = 2, 16
NW = NC * NS
C = 512
NBUF = 4
K = NBUF * (-(-E // (NW * C * NBUF)))
EPAD = NW * K * C
NPAD = 10112
DUMMY = 10000
RPT = NPAD // NS
W = 16
KL = NLAB // NW

_mesh = plsc.VectorSubcoreMesh(
    core_axis_name="c", subcore_axis_name="s", num_cores=NC, num_subcores=NS)


@functools.partial(
    pl.kernel,
    out_type=jax.ShapeDtypeStruct((NC, NPAD, W), jnp.float32),
    mesh=_mesh,
    scratch_types=[
        pltpu.VMEM((K, C), jnp.int32),
        pltpu.VMEM((K, C), jnp.int32),
        pltpu.VMEM((NBUF, C, W), jnp.float32),
        pltpu.VMEM_SHARED((NPAD, W), jnp.float32),
        pltpu.VMEM_SHARED((NPAD, W), jnp.float32),
        [pltpu.SemaphoreType.DMA] * NBUF,
    ],
    compiler_params=pltpu.CompilerParams(use_tc_tiling_on_sc=False),
)
def _segsum_sc(p_hbm, srcs_hbm, dsts_hbm, zeros_hbm, out_hbm,
               src_v, dst_v, rows_v, agg_sh, p_sh, sems):
    cid = lax.axis_index("c")
    sid = lax.axis_index("s")
    wid = sid * NC + cid
    pltpu.sync_copy(srcs_hbm.at[wid], src_v)
    pltpu.sync_copy(dsts_hbm.at[wid], dst_v)
    pltpu.sync_copy(p_hbm.at[pl.ds(sid * RPT, RPT)],
                    p_sh.at[pl.ds(sid * RPT, RPT)])
    pltpu.sync_copy(zeros_hbm.at[pl.ds(sid * RPT, RPT)],
                    agg_sh.at[pl.ds(sid * RPT, RPT)])
    plsc.subcore_barrier()
    for b in range(NBUF):
        pltpu.async_copy(p_sh.at[src_v.at[b]], rows_v.at[b], sems[b])

    def body(g, carry):
        for b in range(NBUF):
            k = g * NBUF + b
            pltpu.make_async_copy(
                p_sh.at[src_v.at[k]], rows_v.at[b], sems[b]).wait()
            pltpu.sync_copy(rows_v.at[b], agg_sh.at[dst_v.at[k]], add=True)

            @pl.when(k + NBUF < K)
            def _():
                pltpu.async_copy(
                    p_sh.at[src_v.at[k + NBUF]], rows_v.at[b], sems[b])
        return carry

    lax.fori_loop(0, K // NBUF, body, 0)
    plsc.subcore_barrier()
    pltpu.sync_copy(agg_sh.at[pl.ds(sid * RPT, RPT)],
                    out_hbm.at[cid, pl.ds(sid * RPT, RPT)])


@functools.partial(
    pl.kernel,
    out_type=jax.ShapeDtypeStruct((NLAB, W), jnp.float32),
    mesh=_mesh,
    scratch_types=[
        pltpu.VMEM((1, KL), jnp.int32),
        pltpu.VMEM((KL, W), jnp.float32),
        pltpu.SemaphoreType.DMA,
    ],
    compiler_params=pltpu.CompilerParams(use_tc_tiling_on_sc=False),
)
def _label_gather_sc(h2_hbm, lbl_hbm, out_hbm, idx_v, rows_v, sem):
    cid = lax.axis_index("c")
    sid = lax.axis_index("s")
    wid = sid * NC + cid
    pltpu.sync_copy(lbl_hbm.at[wid], idx_v)
    pltpu.async_copy(h2_hbm.at[idx_v.at[0]], rows_v, sem).wait()
    pltpu.sync_copy(rows_v, out_hbm.at[pl.ds(wid * KL, KL)])


def _tc0_body(score_ref, wa_ref, wb_ref, sa_ref, p_ref):
    s = score_ref[:]
    sa_ref[:] = jnp.dot(s, wa_ref[:], preferred_element_type=jnp.float32)
    p_ref[:N] = jnp.dot(s, wb_ref[:], preferred_element_type=jnp.float32)
    p_ref[N:] = jnp.zeros((NPAD - N, W), jnp.float32)


def _tc_mid_body(sa_ref, agg_ref, b1_ref, w2_ref, b2_ref, w3_ref, b3_ref,
                 wa_ref, wb_ref, sa_o, p_o):
    agg = agg_ref[0, :N, :] + agg_ref[1, :N, :]
    h = jnp.maximum(sa_ref[:] + agg + b1_ref[:], 0.0)
    h = jnp.maximum(
        jnp.dot(h, w2_ref[:], preferred_element_type=jnp.float32) + b2_ref[:],
        0.0)
    s = jnp.dot(h, w3_ref[:], preferred_element_type=jnp.float32) + b3_ref[:]
    sa_o[:] = jnp.dot(s, wa_ref[:], preferred_element_type=jnp.float32)
    p_o[:N] = jnp.dot(s, wb_ref[:], preferred_element_type=jnp.float32)
    p_o[N:] = jnp.zeros((NPAD - N, W), jnp.float32)


def _tc_last_body(sa_ref, agg_ref, b1_ref, w2_ref, b2_ref, h2_o):
    agg = agg_ref[0, :N, :] + agg_ref[1, :N, :]
    h = jnp.maximum(sa_ref[:] + agg + b1_ref[:], 0.0)
    h2_o[:] = jnp.maximum(
        jnp.dot(h, w2_ref[:], preferred_element_type=jnp.float32) + b2_ref[:],
        0.0)


def _tc_fin_body(hl_ref, w3_ref, b3_ref, g1_ref, g1b_ref, g2_ref, g2b_ref,
                 g3_ref, g3b_ref, out_ref):
    s = jnp.dot(hl_ref[:], w3_ref[:], preferred_element_type=jnp.float32)
    s = s + b3_ref[:]
    h = jnp.maximum(
        jnp.dot(s, g1_ref[:], preferred_element_type=jnp.float32) + g1b_ref[:],
        0.0)
    h = jnp.maximum(
        jnp.dot(h, g2_ref[:], preferred_element_type=jnp.float32) + g2b_ref[:],
        0.0)
    lg = jnp.dot(h, g3_ref[:], preferred_element_type=jnp.float32) + g3b_ref[:]
    out_ref[:] = 1.0 / (1.0 + jnp.exp(-lg))


def _f32(shape):
    return jax.ShapeDtypeStruct(shape, jnp.float32)


_tc0 = pl.pallas_call(_tc0_body, out_shape=(_f32((N, W)), _f32((NPAD, W))))
_tc_mid = pl.pallas_call(
    _tc_mid_body, out_shape=(_f32((N, W)), _f32((NPAD, W))))
_tc_last = pl.pallas_call(_tc_last_body, out_shape=_f32((N, W)))
_tc_fin = pl.pallas_call(_tc_fin_body, out_shape=_f32((NLAB, 1)))


def kernel(score, edges, label_idx, W1, b1, W2, b2, W3, b3,
           G1, g1, G2, g2, G3, g3):
    src = edges[0].astype(jnp.int32)
    dst = edges[1].astype(jnp.int32)
    srcs = jnp.concatenate(
        [src, jnp.zeros((EPAD - E,), jnp.int32)]).reshape(NW, K, C)
    dsts = jnp.concatenate(
        [dst, jnp.full((EPAD - E,), DUMMY, jnp.int32)]).reshape(NW, K, C)
    lbl = label_idx.astype(jnp.int32).reshape(NW, 1, KL)
    zblk = jnp.zeros((NPAD, W), jnp.float32)

    w1a = jnp.pad(W1[:M], ((0, 0), (0, W - 7)))
    w1b = jnp.pad(W1[M:], ((0, 0), (0, W - 7)))
    b1p = jnp.pad(b1, (0, W - 7)).reshape(1, W)
    w2p = jnp.pad(W2, ((0, W - 7), (0, W - 7)))
    b2p = jnp.pad(b2, (0, W - 7)).reshape(1, W)
    w3p = jnp.pad(W3, ((0, W - 7), (0, 0)))
    b3p = b3.reshape(1, M)
    g1p = jnp.pad(G1, ((0, 0), (0, W - 9)))
    g1bp = jnp.pad(g1, (0, W - 9)).reshape(1, W)
    g2p = jnp.pad(G2, ((0, W - 9), (0, W - 9)))
    g2bp = jnp.pad(g2, (0, W - 9)).reshape(1, W)
    g3p = jnp.pad(G3, ((0, W - 9), (0, 0)))
    g3bp = g3.reshape(1, 1)

    sa, p = _tc0(score, w1a, w1b)
    h2 = None
    for t in range(3):
        agg = _segsum_sc(p, srcs, dsts, zblk)
        if t < 2:
            sa, p = _tc_mid(sa, agg, b1p, w2p, b2p, w3p, b3p, w1a, w1b)
        else:
            h2 = _tc_last(sa, agg, b1p, w2p, b2p)

    hl = _label_gather_sc(h2, lbl)
    return _tc_fin(hl, w3p, b3p, g1p, g1bp, g2p, g2bp, g3p, g3bp)


# speedup vs baseline: 1.8881x; 1.2780x over previous
import functools

import jax
import jax.numpy as jnp
from jax import lax
from jax.experimental import pallas as pl
from jax.experimental.pallas import tpu as pltpu
from jax.experimental.pallas import tpu_sc as plsc

N = 10000
M = 128
E = 320000
NLAB = 2048
NC, NS = 2, 16
NW = NC * NS
C = 512
NBUF = 4
K = NBUF * (-(-E // (NW * C * NBUF)))
EPAD = NW * K * C
NPAD = 10240
DUMMY = 10000
RPT = NPAD // NS
W = 16
R8 = NPAD // 8
KL = NLAB // NW

_mesh = plsc.VectorSubcoreMesh(
    core_axis_name="c", subcore_axis_name="s", num_cores=NC, num_subcores=NS)


@functools.partial(
    pl.kernel,
    out_type=jax.ShapeDtypeStruct((NC, NPAD, W), jnp.float32),
    mesh=_mesh,
    scratch_types=[
        pltpu.VMEM((K, C), jnp.int32),
        pltpu.VMEM((K, C), jnp.int32),
        pltpu.VMEM((NBUF, C, W), jnp.float32),
        pltpu.VMEM_SHARED((NPAD, W), jnp.float32),
        pltpu.VMEM_SHARED((NPAD, W), jnp.float32),
        [pltpu.SemaphoreType.DMA] * NBUF,
    ],
    compiler_params=pltpu.CompilerParams(use_tc_tiling_on_sc=False),
)
def _segsum_sc(p_hbm, srcs_hbm, dsts_hbm, zeros_hbm, out_hbm,
               src_v, dst_v, rows_v, agg_sh, p_sh, sems):
    cid = lax.axis_index("c")
    sid = lax.axis_index("s")
    wid = sid * NC + cid
    pltpu.sync_copy(srcs_hbm.at[wid], src_v)
    pltpu.sync_copy(dsts_hbm.at[wid], dst_v)
    pltpu.sync_copy(p_hbm.at[pl.ds(sid * RPT, RPT)],
                    p_sh.at[pl.ds(sid * RPT, RPT)])
    pltpu.sync_copy(zeros_hbm.at[pl.ds(sid * RPT, RPT)],
                    agg_sh.at[pl.ds(sid * RPT, RPT)])
    plsc.subcore_barrier()
    for b in range(NBUF):
        pltpu.async_copy(p_sh.at[src_v.at[b]], rows_v.at[b], sems[b])

    def body(g, carry):
        for b in range(NBUF):
            k = g * NBUF + b
            pltpu.make_async_copy(
                p_sh.at[src_v.at[k]], rows_v.at[b], sems[b]).wait()
            pltpu.sync_copy(rows_v.at[b], agg_sh.at[dst_v.at[k]], add=True)

            @pl.when(k + NBUF < K)
            def _():
                pltpu.async_copy(
                    p_sh.at[src_v.at[k + NBUF]], rows_v.at[b], sems[b])
        return carry

    lax.fori_loop(0, K // NBUF, body, 0)
    plsc.subcore_barrier()
    pltpu.sync_copy(agg_sh.at[pl.ds(sid * RPT, RPT)],
                    out_hbm.at[cid, pl.ds(sid * RPT, RPT)])


@functools.partial(
    pl.kernel,
    out_type=jax.ShapeDtypeStruct((NLAB, W), jnp.float32),
    mesh=_mesh,
    scratch_types=[
        pltpu.VMEM((1, KL), jnp.int32),
        pltpu.VMEM((KL, W), jnp.float32),
        pltpu.SemaphoreType.DMA,
    ],
    compiler_params=pltpu.CompilerParams(use_tc_tiling_on_sc=False),
)
def _label_gather_sc(h2_hbm, lbl_hbm, out_hbm, idx_v, rows_v, sem):
    cid = lax.axis_index("c")
    sid = lax.axis_index("s")
    wid = sid * NC + cid
    pltpu.sync_copy(lbl_hbm.at[wid], idx_v)
    pltpu.async_copy(h2_hbm.at[idx_v.at[0]], rows_v, sem).wait()
    pltpu.sync_copy(rows_v, out_hbm.at[pl.ds(wid * KL, KL)])


def _dot(a, b):
    return jnp.dot(a, b, preferred_element_type=jnp.float32)


def _slotmap(x, w):
    win = w.shape[0]
    return jnp.concatenate(
        [_dot(x[:, a * win:(a + 1) * win], w) for a in range(8)], axis=1)


def _tc0_body(score_ref, wa_ref, wb_ref, sa_ref, p_ref):
    s = score_ref[:]
    sa_ref[:] = _slotmap(s, wa_ref[:])
    p_ref[:] = _slotmap(s, wb_ref[:])


def _tc_mid_body(sa_ref, agg_ref, b1_ref, w2_ref, b2_ref, w3_ref, b3_ref,
                 wa_ref, wb_ref, sa_o, p_o):
    h = jnp.maximum(sa_ref[:] + agg_ref[0] + agg_ref[1] + b1_ref[:], 0.0)
    h = jnp.maximum(_slotmap(h, w2_ref[:]) + b2_ref[:], 0.0)
    s = _slotmap(h, w3_ref[:]) + b3_ref[:]
    sa_o[:] = _slotmap(s, wa_ref[:])
    p_o[:] = _slotmap(s, wb_ref[:])


def _tc_last_body(sa_ref, agg_ref, b1_ref, w2_ref, b2_ref, h2_o):
    h = jnp.maximum(sa_ref[:] + agg_ref[0] + agg_ref[1] + b1_ref[:], 0.0)
    h2_o[:] = jnp.maximum(_slotmap(h, w2_ref[:]) + b2_ref[:], 0.0)


def _tc_fin_body(hl_ref, w3_ref, b3_ref, g1_ref, g1b_ref, g2_ref, g2b_ref,
                 g3_ref, g3b_ref, out_ref):
    s = _slotmap(hl_ref[:], w3_ref[:]) + b3_ref[:]
    h = jnp.maximum(_slotmap(s, g1_ref[:]) + g1b_ref[:], 0.0)
    h = jnp.maximum(_slotmap(h, g2_ref[:]) + g2b_ref[:], 0.0)
    lg = _slotmap(h, g3_ref[:]) + g3b_ref[:]
    out_ref[:] = 1.0 / (1.0 + jnp.exp(-lg))


def _f32(shape):
    return jax.ShapeDtypeStruct(shape, jnp.float32)


_tc0 = pl.pallas_call(_tc0_body, out_shape=(_f32((R8, M)), _f32((R8, M))))
_tc_mid = pl.pallas_call(_tc_mid_body, out_shape=(_f32((R8, M)), _f32((R8, M))))
_tc_last = pl.pallas_call(_tc_last_body, out_shape=_f32((R8, M)))
_tc_fin = pl.pallas_call(_tc_fin_body, out_shape=_f32((NLAB // 8, 8)))


def _pad16(w):
    return jnp.pad(w, ((0, W - w.shape[0]), (0, W - w.shape[1])))


def _tile8(v):
    return jnp.tile(jnp.pad(v, (0, W - v.shape[0])), 8).reshape(1, 8 * W)


def kernel(score, edges, label_idx, W1, b1, W2, b2, W3, b3,
           G1, g1, G2, g2, G3, g3):
    src = edges[0].astype(jnp.int32)
    dst = edges[1].astype(jnp.int32)
    srcs = jnp.concatenate(
        [src, jnp.zeros((EPAD - E,), jnp.int32)]).reshape(NW, K, C)
    dsts = jnp.concatenate(
        [dst, jnp.full((EPAD - E,), DUMMY, jnp.int32)]).reshape(NW, K, C)
    lbl = label_idx.astype(jnp.int32).reshape(NW, 1, KL)
    zblk = jnp.zeros((NPAD, W), jnp.float32)
    score8 = jnp.pad(score, ((0, NPAD - N), (0, 0))).reshape(R8, 8 * M)

    W1a, W1b = W1[:M], W1[M:]
    wa = jnp.pad(W1a, ((0, 0), (0, W - 7)))
    wb = jnp.pad(W1b, ((0, 0), (0, W - 7)))
    w2p = _pad16(W2)
    w3p = jnp.pad(W3, ((0, W - 7), (0, 0)))
    g1p = jnp.pad(G1, ((0, 0), (0, W - 9)))
    g2p = _pad16(G2)
    g3p = jnp.pad(G3, ((0, W - 9), (0, 0)))
    b1t, b2t = _tile8(b1), _tile8(b2)
    b3t = jnp.tile(b3, 8).reshape(1, 8 * M)
    g1t, g2t = _tile8(g1), _tile8(g2)
    g3t = jnp.tile(g3, 8).reshape(1, 8)

    sa, p = _tc0(score8, wa, wb)
    h2 = None
    for t in range(3):
        agg = _segsum_sc(p.reshape(NPAD, W), srcs, dsts, zblk)
        agg8 = agg.reshape(NC, R8, 8 * W)
        if t < 2:
            sa, p = _tc_mid(sa, agg8, b1t, w2p, b2t, w3p, b3t, wa, wb)
        else:
            h2 = _tc_last(sa, agg8, b1t, w2p, b2t)

    hl = _label_gather_sc(h2.reshape(NPAD, W), lbl)
    pred = _tc_fin(hl.reshape(NLAB // 8, 8 * W),
                   w3p, b3t, g1p, g1t, g2p, g2t, g3p, g3t)
    return pred.reshape(NLAB, 1)
